# Initial kernel scaffold; baseline (speedup 1.0000x reference)
import jax
import jax.numpy as jnp
from jax.experimental import pallas as pl


def kernel(h, edge_index, edge_sh, edge_radial_embedding, W1, b1, W2, b2, W3, b3, W4, b4, W_sh, W_mid, W_sc, U1, U2, U3, gamma, beta):
    raise NotImplementedError("write your pallas kernel here")


# speedup vs baseline: 2.2444x; 2.2444x over previous
import functools

import jax
import jax.numpy as jnp
from jax import lax
from jax.experimental import pallas as pl
from jax.experimental.pallas import tpu as pltpu
from jax.experimental.pallas import tpu_sc as plsc

N = 10000
E = 320000
C = 128
SH = 9
RBF = 16
H = 64

NC = 2
NS = 16
L = 16
NW = NC * NS
EPW = E // NW
B = 80
NB = EPW // B
NCHUNK = N // B

BE = 2048


def _edge_w_body(rbf_ref, sh_ref, w1, b1, w2, b2, w3, b3, w4, b4, wsh,
                 out_ref):
    x = rbf_ref[...]
    x = jax.nn.silu(x @ w1[...] + b1[...])
    x = jax.nn.silu(x @ w2[...] + b2[...])
    x = jax.nn.silu(x @ w3[...] + b3[...])
    tp = x @ w4[...] + b4[...]
    shp = sh_ref[...] @ wsh[...]
    out_ref[...] = tp * shp


def _edge_w(rbf, sh, W1, b1, W2, b2, W3, b3, W4, b4, W_sh):
    grid = (E // BE,)
    full = lambda r, c: pl.BlockSpec((r, c), lambda i: (0, 0))
    return pl.pallas_call(
        _edge_w_body,
        grid=grid,
        in_specs=[
            pl.BlockSpec((BE, RBF), lambda i: (i, 0)),
            pl.BlockSpec((BE, SH), lambda i: (i, 0)),
            full(RBF, H), full(1, H),
            full(H, H), full(1, H),
            full(H, H), full(1, H),
            full(H, C), full(1, C),
            full(SH, C),
        ],
        out_specs=pl.BlockSpec((BE, C), lambda i: (i, 0)),
        out_shape=jax.ShapeDtypeStruct((E, C), jnp.float32),
    )(rbf, sh, W1, b1.reshape(1, H), W2, b2.reshape(1, H),
      W3, b3.reshape(1, H), W4, b4.reshape(1, C), W_sh)


def _sc_body(h_hbm, w_hbm, src_hbm, dst_hbm, out_hbm,
             srcv, dstv, hrow, wrow, agg, gsem):
    c = lax.axis_index("c")
    s = lax.axis_index("s")
    wid = c * NS + s

    zeros = jnp.zeros((L,), jnp.float32)

    def _zero_row(e, _):
        for t in range(C // L):
            hrow[e, pl.ds(t * L, L)] = zeros
        return 0

    lax.fori_loop(0, B, _zero_row, 0)
    for k in range((NCHUNK + NS - 1) // NS):
        chunk = s + k * NS

        @pl.when(chunk < NCHUNK)
        def _():
            pltpu.sync_copy(hrow, agg.at[pl.ds(chunk * B, B)])

    plsc.subcore_barrier()

    def _batch(j, _):
        base = wid * EPW + j * B
        pltpu.sync_copy(src_hbm.at[wid, j], srcv)
        pltpu.sync_copy(dst_hbm.at[wid, j], dstv)
        gcp = pltpu.async_copy(h_hbm.at[srcv], hrow, gsem)
        pltpu.sync_copy(w_hbm.at[pl.ds(base, B)], wrow)
        gcp.wait()

        def _mul_row(e, _):
            for t in range(C // L):
                sl = pl.ds(t * L, L)
                hrow[e, sl] = hrow[e, sl] * wrow[e, sl]
            return 0

        lax.fori_loop(0, B, _mul_row, 0)
        pltpu.sync_copy(hrow, agg.at[dstv], add=True)
        return 0

    lax.fori_loop(0, NB, _batch, 0)
    plsc.subcore_barrier()

    for k in range((NCHUNK + NS - 1) // NS):
        chunk = s + k * NS

        @pl.when(chunk < NCHUNK)
        def _():
            pltpu.sync_copy(agg.at[pl.ds(chunk * B, B)],
                            out_hbm.at[c, pl.ds(chunk * B, B)])


def _sc_aggregate(h, w, src3, dst3):
    mesh = plsc.VectorSubcoreMesh(core_axis_name="c", subcore_axis_name="s")
    f = functools.partial(
        pl.kernel,
        out_type=jax.ShapeDtypeStruct((NC, N, C), jnp.float32),
        mesh=mesh,
        scratch_types=[
            pltpu.VMEM((B,), jnp.int32),
            pltpu.VMEM((B,), jnp.int32),
            pltpu.VMEM((B, C), jnp.float32),
            pltpu.VMEM((B, C), jnp.float32),
            pltpu.VMEM_SHARED((N, C), jnp.float32),
            pltpu.SemaphoreType.DMA,
        ],
    )(_sc_body)
    return f(h, w, src3, dst3)


def _node_body(p_ref, h_ref, wmid, wsc, u1, u2, u3, gamma, beta, out_ref):
    agg = p_ref[0] + p_ref[1]
    a = agg @ wmid[...]
    sc = h_ref[...] @ wsc[...]
    msg = a @ u1[...] + (a * a) @ u2[...] + (a * a * a) @ u3[...] + sc
    mean = jnp.mean(msg, axis=0, keepdims=True)
    var = jnp.mean((msg - mean) * (msg - mean), axis=0, keepdims=True)
    out_ref[...] = ((msg - mean) * lax.rsqrt(var + 1e-5) * gamma[...]
                    + beta[...])


def _node_tail(partials, h, W_mid, W_sc, U1, U2, U3, gamma, beta):
    return pl.pallas_call(
        _node_body,
        out_shape=jax.ShapeDtypeStruct((N, C), jnp.float32),
    )(partials, h, W_mid, W_sc, U1, U2, U3,
      gamma.reshape(1, C), beta.reshape(1, C))


def kernel(h, edge_index, edge_sh, edge_radial_embedding, W1, b1, W2, b2,
           W3, b3, W4, b4, W_sh, W_mid, W_sc, U1, U2, U3, gamma, beta):
    w = _edge_w(edge_radial_embedding, edge_sh,
                W1, b1, W2, b2, W3, b3, W4, b4, W_sh)
    src3 = edge_index[0].reshape(NW, NB, B)
    dst3 = edge_index[1].reshape(NW, NB, B)
    partials = _sc_aggregate(h, w, src3, dst3)
    return _node_tail(partials, h, W_mid, W_sc, U1, U2, U3, gamma, beta)


# speedup vs baseline: 2.9683x; 1.3225x over previous
import functools

import jax
import jax.numpy as jnp
from jax import lax
from jax.experimental import pallas as pl
from jax.experimental.pallas import tpu as pltpu
from jax.experimental.pallas import tpu_sc as plsc

N = 10000
E = 320000
C = 128
SH = 9
RBF = 16
H = 64

NC = 2
NS = 16
L = 16
NW = NC * NS
B = 128
NBAT = E // B
NBF = NBAT // NW
NLEFT = NBAT - NBF * NW
ZB = 80
NCHUNK = N // ZB

BE = 2048


def _edge_w_body(rbf_ref, sh_ref, w1, b1, w2, b2, w3, b3, w4, b4, wsh,
                 out_ref):
    x = rbf_ref[...]
    x = jax.nn.silu(x @ w1[...] + b1[...])
    x = jax.nn.silu(x @ w2[...] + b2[...])
    x = jax.nn.silu(x @ w3[...] + b3[...])
    tp = x @ w4[...] + b4[...]
    shp = sh_ref[...] @ wsh[...]
    out_ref[...] = tp * shp


def _edge_w(rbf, sh, W1, b1, W2, b2, W3, b3, W4, b4, W_sh):
    grid = (E // BE,)
    full = lambda r, c: pl.BlockSpec((r, c), lambda i: (0, 0))
    return pl.pallas_call(
        _edge_w_body,
        grid=grid,
        in_specs=[
            pl.BlockSpec((BE, RBF), lambda i: (i, 0)),
            pl.BlockSpec((BE, SH), lambda i: (i, 0)),
            full(RBF, H), full(1, H),
            full(H, H), full(1, H),
            full(H, H), full(1, H),
            full(H, C), full(1, C),
            full(SH, C),
        ],
        out_specs=pl.BlockSpec((BE, C), lambda i: (i, 0)),
        out_shape=jax.ShapeDtypeStruct((E, C), jnp.float32),
    )(rbf, sh, W1, b1.reshape(1, H), W2, b2.reshape(1, H),
      W3, b3.reshape(1, H), W4, b4.reshape(1, C), W_sh)


def _sc_body(h_hbm, w_hbm, ei_hbm, out_hbm,
             srcv, dstv, hrow, gsem,
             srcv2, dstv2, hrow2, gsem2, wrow, wsem, agg):
    c = lax.axis_index("c")
    s = lax.axis_index("s")
    wid = c * NS + s
    bufs = ((srcv, dstv, hrow, gsem), (srcv2, dstv2, hrow2, gsem2))

    def _fetch(b, buf):
        sv, dv, hr, gs = buf
        base = b * B
        pltpu.sync_copy(ei_hbm.at[0, pl.ds(base, B)], sv)
        pltpu.sync_copy(ei_hbm.at[1, pl.ds(base, B)], dv)
        pltpu.async_copy(h_hbm.at[sv], hr, gs)

    def _wload(b):
        pltpu.async_copy(w_hbm.at[pl.ds(b * B, B)], wrow, wsem)

    def _wwait(b):
        pltpu.make_async_copy(w_hbm.at[pl.ds(b * B, B)], wrow, wsem).wait()

    def _mul(hr):
        def _mul_row(e, _):
            for t in range(C // L):
                sl = pl.ds(t * L, L)
                hr[e, sl] = hr[e, sl] * wrow[e, sl]
            return 0

        lax.fori_loop(0, B, _mul_row, 0)

    def _compute(b, buf, next_w):
        sv, dv, hr, gs = buf
        pltpu.make_async_copy(h_hbm.at[sv], hr, gs).wait()
        _wwait(b)
        _mul(hr)
        if next_w is not None:
            next_w()
        pltpu.sync_copy(hr, agg.at[dv], add=True)

    zeros = jnp.zeros((L,), jnp.float32)

    def _zero_row(e, _):
        for t in range(C // L):
            hrow[e, pl.ds(t * L, L)] = zeros
        return 0

    lax.fori_loop(0, ZB, _zero_row, 0)
    for k in range((NCHUNK + NS - 1) // NS):
        chunk = s + k * NS

        @pl.when(chunk < NCHUNK)
        def _():
            pltpu.sync_copy(hrow.at[pl.ds(0, ZB)], agg.at[pl.ds(chunk * ZB, ZB)])

    plsc.subcore_barrier()

    b0 = wid * NBF
    npair = NBF // 2
    _wload(b0)
    _fetch(b0, bufs[0])

    def _pair(m, _):
        j0 = b0 + 2 * m
        _fetch(j0 + 1, bufs[1])
        _compute(j0, bufs[0], lambda: _wload(j0 + 1))

        @pl.when(m < npair - 1)
        def _():
            _fetch(j0 + 2, bufs[0])

        def _next_w():
            @pl.when(m < npair - 1)
            def _():
                _wload(j0 + 2)

        _compute(j0 + 1, bufs[1], _next_w)
        return 0

    lax.fori_loop(0, npair, _pair, 0)

    @pl.when(wid < NLEFT)
    def _():
        bl = NW * NBF + wid
        _wload(bl)
        _fetch(bl, bufs[0])
        _compute(bl, bufs[0], None)

    plsc.subcore_barrier()

    for k in range((NCHUNK + NS - 1) // NS):
        chunk = s + k * NS

        @pl.when(chunk < NCHUNK)
        def _():
            pltpu.sync_copy(agg.at[pl.ds(chunk * ZB, ZB)],
                            out_hbm.at[c, pl.ds(chunk * ZB, ZB)])


def _sc_aggregate(h, w, edge_index):
    mesh = plsc.VectorSubcoreMesh(core_axis_name="c", subcore_axis_name="s")
    f = functools.partial(
        pl.kernel,
        out_type=jax.ShapeDtypeStruct((NC, N, C), jnp.float32),
        mesh=mesh,
        scratch_types=[
            pltpu.VMEM((B,), jnp.int32),
            pltpu.VMEM((B,), jnp.int32),
            pltpu.VMEM((B, C), jnp.float32),
            pltpu.SemaphoreType.DMA,
            pltpu.VMEM((B,), jnp.int32),
            pltpu.VMEM((B,), jnp.int32),
            pltpu.VMEM((B, C), jnp.float32),
            pltpu.SemaphoreType.DMA,
            pltpu.VMEM((B, C), jnp.float32),
            pltpu.SemaphoreType.DMA,
            pltpu.VMEM_SHARED((N, C), jnp.float32),
        ],
    )(_sc_body)
    return f(h, w, edge_index)


def _node_body(p_ref, h_ref, wmid, wsc, u1, u2, u3, gamma, beta, out_ref):
    agg = p_ref[0] + p_ref[1]
    a = agg @ wmid[...]
    sc = h_ref[...] @ wsc[...]
    msg = a @ u1[...] + (a * a) @ u2[...] + (a * a * a) @ u3[...] + sc
    mean = jnp.mean(msg, axis=0, keepdims=True)
    var = jnp.mean((msg - mean) * (msg - mean), axis=0, keepdims=True)
    out_ref[...] = ((msg - mean) * lax.rsqrt(var + 1e-5) * gamma[...]
                    + beta[...])


def _node_tail(partials, h, W_mid, W_sc, U1, U2, U3, gamma, beta):
    return pl.pallas_call(
        _node_body,
        out_shape=jax.ShapeDtypeStruct((N, C), jnp.float32),
    )(partials, h, W_mid, W_sc, U1, U2, U3,
      gamma.reshape(1, C), beta.reshape(1, C))


def kernel(h, edge_index, edge_sh, edge_radial_embedding, W1, b1, W2, b2,
           W3, b3, W4, b4, W_sh, W_mid, W_sc, U1, U2, U3, gamma, beta):
    w = _edge_w(edge_radial_embedding, edge_sh,
                W1, b1, W2, b2, W3, b3, W4, b4, W_sh)
    partials = _sc_aggregate(h, w, edge_index)
    return _node_tail(partials, h, W_mid, W_sc, U1, U2, U3, gamma, beta)


# speedup vs baseline: 4.5987x; 1.5493x over previous
import functools

import numpy as np

import jax
import jax.numpy as jnp
from jax import lax
from jax.experimental import pallas as pl
from jax.experimental.pallas import tpu as pltpu
from jax.experimental.pallas import tpu_sc as plsc

N = 10000
E = 320000
C = 128
SH = 9
RBF = 16
H = 64

NC = 2
NS = 16
L = 16
NW = NC * NS
B = 128
E2 = E // 2
NPAIR = E2 // B
NPF = NPAIR // NW
NLEFT = NPAIR - NPF * NW
ZB = 80
NCHUNK = N // ZB

BE = 3200


_DNT = (((0,), (0,)), ((), ()))


def _edge_w_body(rbft_lo, sht_lo, rbft_hi, sht_hi,
                 w1, b1, w2, b2, w3, b3, w4, b4, wsh,
                 out_ref, shpad_ref):
    @pl.when(pl.program_id(0) == 0)
    def _():
        shpad_ref[...] = jnp.zeros((2 * SH - 2, BE), jnp.float32)

    def half_w(rbft_ref, sht_ref):
        shpad_ref[pl.ds(0, SH), :] = sht_ref[...]
        x = lax.dot_general(rbft_ref[...], w1[...], _DNT) + b1[...]
        x = jax.nn.silu(x)
        x = jax.nn.silu(x @ w2[...] + b2[...])
        x = jax.nn.silu(x @ w3[...] + b3[...])
        tp = x @ w4[...] + b4[...]
        shp = lax.dot_general(shpad_ref[...], wsh[...], _DNT)
        return tp * shp

    w_lo = half_w(rbft_lo, sht_lo)
    w_hi = half_w(rbft_hi, sht_hi)
    half_ulp = jnp.int32(0x8000)
    lo_b = lax.shift_right_logical(
        lax.bitcast_convert_type(w_lo, jnp.int32) + half_ulp, 16)
    hi_b = ((lax.bitcast_convert_type(w_hi, jnp.int32) + half_ulp)
            & jnp.int32(np.int32(-65536)))
    out_ref[...] = lo_b | hi_b


def _edge_w(rbft, sht, W1, b1, W2, b2, W3, b3, W4, b4, W_sh):
    grid = (E2 // BE,)
    nblk = E2 // BE
    full = lambda r, c: pl.BlockSpec((r, c), lambda i: (0, 0))
    return pl.pallas_call(
        _edge_w_body,
        grid=grid,
        in_specs=[
            pl.BlockSpec((RBF, BE), lambda i: (0, i)),
            pl.BlockSpec((SH, BE), lambda i: (0, i)),
            pl.BlockSpec((RBF, BE), lambda i: (0, i + nblk)),
            pl.BlockSpec((SH, BE), lambda i: (0, i + nblk)),
            full(RBF, H), full(1, H),
            full(H, H), full(1, H),
            full(H, H), full(1, H),
            full(H, C), full(1, C),
            full(2 * SH - 2, C),
        ],
        out_specs=pl.BlockSpec((BE, C), lambda i: (i, 0)),
        out_shape=jax.ShapeDtypeStruct((E2, C), jnp.int32),
        scratch_shapes=[pltpu.VMEM((2 * SH - 2, BE), jnp.float32)],
    )(rbft, sht, rbft, sht, W1, b1.reshape(1, H), W2, b2.reshape(1, H),
      W3, b3.reshape(1, H), W4, b4.reshape(1, C),
      jnp.zeros((2 * SH - 2, C), W_sh.dtype).at[:SH].set(W_sh))


def _sc_body(h_hbm, w_hbm, ei_hbm, out_hbm,
             srcv, dstv, hrow, gsem,
             srcv2, dstv2, hrow2, gsem2, wrow, wsem, agg):
    c = lax.axis_index("c")
    s = lax.axis_index("s")
    wid = c * NS + s
    bufs = ((srcv, dstv, hrow, gsem), (srcv2, dstv2, hrow2, gsem2))

    def _fetch(b, buf):
        sv, dv, hr, gs = buf
        base = b * B
        pltpu.sync_copy(ei_hbm.at[0, pl.ds(base, B)], sv)
        pltpu.sync_copy(ei_hbm.at[1, pl.ds(base, B)], dv)
        pltpu.async_copy(h_hbm.at[sv], hr, gs)

    def _wload(p):
        pltpu.async_copy(w_hbm.at[pl.ds(p * B, B)], wrow, wsem)

    def _wwait(p):
        pltpu.make_async_copy(w_hbm.at[pl.ds(p * B, B)], wrow, wsem).wait()

    def _mul(hr, hi_half):
        def _mul_row(e, _):
            for g in range(C // L):
                sl = pl.ds(L * g, L)
                wi = wrow[e, sl]
                if hi_half:
                    wv = lax.bitcast_convert_type(
                        wi & jnp.int32(np.int32(-65536)), jnp.float32)
                else:
                    wv = lax.bitcast_convert_type(wi << 16, jnp.float32)
                hr[e, sl] = hr[e, sl] * wv
            return 0

        lax.fori_loop(0, B, _mul_row, 0)

    def _compute(p, buf, hi_half, wait_w, next_w):
        sv, dv, hr, gs = buf
        pltpu.make_async_copy(h_hbm.at[sv], hr, gs).wait()
        if wait_w:
            _wwait(p)
        _mul(hr, hi_half)
        if next_w is not None:
            next_w()
        pltpu.sync_copy(hr, agg.at[dv], add=True)

    zeros = jnp.zeros((L,), jnp.float32)

    def _zero_row(e, _):
        for t in range(C // L):
            hrow[e, pl.ds(t * L, L)] = zeros
        return 0

    lax.fori_loop(0, ZB, _zero_row, 0)
    for k in range((NCHUNK + NS - 1) // NS):
        chunk = s + k * NS

        @pl.when(chunk < NCHUNK)
        def _():
            pltpu.sync_copy(hrow.at[pl.ds(0, ZB)], agg.at[pl.ds(chunk * ZB, ZB)])

    plsc.subcore_barrier()

    p0 = wid * NPF
    _wload(p0)
    _fetch(p0, bufs[0])

    def _pair(m, _):
        p = p0 + m
        _fetch(p + NPAIR, bufs[1])
        _compute(p, bufs[0], False, True, None)

        @pl.when(m < NPF - 1)
        def _():
            _fetch(p + 1, bufs[0])

        def _next_w():
            @pl.when(m < NPF - 1)
            def _():
                _wload(p + 1)

        _compute(p + NPAIR, bufs[1], True, False, _next_w)
        return 0

    lax.fori_loop(0, NPF, _pair, 0)

    @pl.when(wid < NLEFT)
    def _():
        pleft = NW * NPF + wid
        _wload(pleft)
        _fetch(pleft, bufs[0])
        _compute(pleft, bufs[0], False, True, None)
        _fetch(pleft + NPAIR, bufs[1])
        _compute(pleft + NPAIR, bufs[1], True, False, None)

    plsc.subcore_barrier()

    for k in range((NCHUNK + NS - 1) // NS):
        chunk = s + k * NS

        @pl.when(chunk < NCHUNK)
        def _():
            pltpu.sync_copy(agg.at[pl.ds(chunk * ZB, ZB)],
                            out_hbm.at[c, pl.ds(chunk * ZB, ZB)])


def _sc_aggregate(h, w, edge_index):
    mesh = plsc.VectorSubcoreMesh(core_axis_name="c", subcore_axis_name="s")
    f = functools.partial(
        pl.kernel,
        out_type=jax.ShapeDtypeStruct((NC, N, C), jnp.float32),
        mesh=mesh,
        scratch_types=[
            pltpu.VMEM((B,), jnp.int32),
            pltpu.VMEM((B,), jnp.int32),
            pltpu.VMEM((B, C), jnp.float32),
            pltpu.SemaphoreType.DMA,
            pltpu.VMEM((B,), jnp.int32),
            pltpu.VMEM((B,), jnp.int32),
            pltpu.VMEM((B, C), jnp.float32),
            pltpu.SemaphoreType.DMA,
            pltpu.VMEM((B, C), jnp.int32),
            pltpu.SemaphoreType.DMA,
            pltpu.VMEM_SHARED((N, C), jnp.float32),
        ],
    )(_sc_body)
    return f(h, w, edge_index)


def _node_body(p_ref, h_ref, wmid, wsc, u1, u2, u3, gamma, beta, out_ref):
    agg = p_ref[0] + p_ref[1]
    a = agg @ wmid[...]
    sc = h_ref[...] @ wsc[...]
    msg = a @ u1[...] + (a * a) @ u2[...] + (a * a * a) @ u3[...] + sc
    mean = jnp.mean(msg, axis=0, keepdims=True)
    var = jnp.mean((msg - mean) * (msg - mean), axis=0, keepdims=True)
    out_ref[...] = ((msg - mean) * lax.rsqrt(var + 1e-5) * gamma[...]
                    + beta[...])


def _node_tail(partials, h, W_mid, W_sc, U1, U2, U3, gamma, beta):
    return pl.pallas_call(
        _node_body,
        out_shape=jax.ShapeDtypeStruct((N, C), jnp.float32),
    )(partials, h, W_mid, W_sc, U1, U2, U3,
      gamma.reshape(1, C), beta.reshape(1, C))


def kernel(h, edge_index, edge_sh, edge_radial_embedding, W1, b1, W2, b2,
           W3, b3, W4, b4, W_sh, W_mid, W_sc, U1, U2, U3, gamma, beta):
    w = _edge_w(edge_radial_embedding.T, edge_sh.T,
                W1, b1, W2, b2, W3, b3, W4, b4, W_sh)
    partials = _sc_aggregate(h, w, edge_index)
    return _node_tail(partials, h, W_mid, W_sc, U1, U2, U3, gamma, beta)
